# baseline (speedup 1.0000x reference)
import functools
import math

import jax
import jax.numpy as jnp
from jax import lax
from jax.experimental import pallas as pl
from jax.experimental.pallas import tpu as pltpu
from jax.experimental.pallas import tpu_sc as plsc

_EPS = 1e-6
_LOG_EPS = math.log(_EPS)
_LOG_1MEPS = math.log1p(-_EPS)
_K = 1000
_HARD_WEIGHT = 0.01
_INTERPRET = False

_ROWS_PER_BLOCK = 8
_N = 100000
_CHUNK = 10000
_NCHUNK = _N // _CHUNK
_NBINS = 1024
_ROWS_PER_WORKER = 2

_C0 = _EPS * _LOG_EPS + (1.0 - _EPS) * _LOG_1MEPS


def _loss_body(x_ref, z_ref, t_ref, out_ref):
    x = x_ref[...]
    z = z_ref[...]
    t = t_ref[...][:, :1]
    ux = jnp.exp(-jnp.abs(x))
    sp_x = jnp.maximum(x, 0.0) + jnp.log1p(ux)
    uz = jnp.exp(-jnp.abs(z))
    sp_z = jnp.maximum(z, 0.0) + jnp.log1p(uz)
    vz = 1.0 / (1.0 + uz)
    s_z = jnp.where(z >= 0.0, vz, uz * vz)
    ls = sp_x - sp_z + s_z * (z - x)
    col = lax.broadcasted_iota(jnp.int32, x.shape, 1)
    lh = (_C0 + sp_x - _EPS * x) - jnp.where(
        col == t, (1.0 - 2.0 * _EPS) * x, 0.0)
    out_ref[...] = jnp.concatenate(
        [jnp.abs(jnp.maximum(ls, 0.0)), jnp.abs(jnp.maximum(lh, 0.0))],
        axis=0)


def _sc_select_body(l_hbm, out_hbm, buf0, buf1, cnt_h, sum_h, outv,
                    sem0, sem1):
    w = lax.axis_index("s") * 2 + lax.axis_index("c")
    lane = lax.iota(jnp.int32, 16)
    ones_i = jnp.ones((16,), jnp.int32)
    zer_i = jnp.zeros((16,), jnp.int32)
    zer_f = jnp.zeros((16,), jnp.float32)

    def hist_pass(base, b_hi):
        def clr(i):
            cnt_h[pl.ds(i * 16, 16)] = zer_i
            if b_hi is not None:
                sum_h[pl.ds(i * 16, 16)] = zer_f

        plsc.parallel_loop(0, _NBINS + 1, unroll=8)(clr)

        bufs = (buf0, buf1)
        sems = (sem0, sem1)
        handles = [None] * _NCHUNK
        handles[0] = pltpu.async_copy(
            l_hbm.at[pl.ds(base, _CHUNK)], bufs[0], sems[0])
        for c in range(_NCHUNK):
            if c + 1 < _NCHUNK:
                handles[c + 1] = pltpu.async_copy(
                    l_hbm.at[pl.ds(base + (c + 1) * _CHUNK, _CHUNK)],
                    bufs[(c + 1) % 2], sems[(c + 1) % 2])
            handles[c].wait()
            cur = bufs[c % 2]

            def scat(j):
                v = cur[pl.ds(j * 16, 16)]
                bits = lax.bitcast_convert_type(v, jnp.int32)
                if b_hi is None:
                    bn = lax.shift_right_logical(bits, 21) & 0x3FF
                    plsc.addupdate_scatter(cnt_h, [bn * 16 + lane], ones_i)
                else:
                    b1v = lax.shift_right_logical(bits, 21) & 0x3FF
                    m_eq = b1v == b_hi
                    m_hi = b1v > b_hi
                    bn = lax.shift_right_logical(bits, 11) & 0x3FF
                    bx = jnp.where(m_hi, jnp.int32(_NBINS), bn)
                    idx = bx * 16 + lane
                    plsc.addupdate_scatter(cnt_h, [idx], ones_i, mask=m_eq)
                    plsc.addupdate_scatter(sum_h, [idx], v,
                                           mask=m_eq | m_hi)

            plsc.parallel_loop(0, _CHUNK // 16, unroll=8)(scat)

    def group_tot(g):
        def acc(i, carry):
            cc, ss = carry
            off = (g * 16 + i) * 16
            return (cc + cnt_h[pl.ds(off, 16)], ss + sum_h[pl.ds(off, 16)])

        cc, ss = lax.fori_loop(0, 16, acc, (zer_i, zer_f))
        return jnp.sum(cc), jnp.sum(ss)

    def bin_tot(b):
        off = b * 16
        return jnp.sum(cnt_h[pl.ds(off, 16)]), jnp.sum(sum_h[pl.ds(off, 16)])

    def find_bin(k_rem):
        def gcond(st):
            g, ca, sa, cg, sg = st
            return (g > 0) & (ca + cg < k_rem)

        def gbody(st):
            g, ca, sa, cg, sg = st
            cn, sn = group_tot(g - 1)
            return (g - 1, ca + cg, sa + sg, cn, sn)

        cg0, sg0 = group_tot(63)
        g, ca, sa, cg, sg = lax.while_loop(
            gcond, gbody, (jnp.int32(63), jnp.int32(0), jnp.float32(0.0),
                           cg0, sg0))

        def bcond(st):
            b, ca, sa, cb, sb = st
            return (b > 0) & (ca + cb < k_rem)

        def bbody(st):
            b, ca, sa, cb, sb = st
            cn, sn = bin_tot(g * 16 + b - 1)
            return (b - 1, ca + cb, sa + sb, cn, sn)

        cb0, sb0 = bin_tot(g * 16 + 15)
        b, ca, sa, cb, sb = lax.while_loop(
            bcond, bbody, (jnp.int32(15), ca, sa, cb0, sb0))
        return g * 16 + b, ca, sa

    def do_row(r, acc):
        base = (w * _ROWS_PER_WORKER + r) * _N
        hist_pass(base, None)
        b1, c1, _ = find_bin(jnp.int32(_K))
        k2 = jnp.int32(_K) - c1
        hist_pass(base, b1)
        b2, c2, s2 = find_bin(k2)
        s_hi = jnp.sum(sum_h[pl.ds(_NBINS * 16, 16)])
        t_bits = (b1 << 21) | (b2 << 11) | (1 << 10)
        t_hat = jnp.max(lax.bitcast_convert_type(
            jnp.full((16,), t_bits, jnp.int32), jnp.float32))
        rs = s_hi + s2 + (k2 - c2).astype(jnp.float32) * t_hat
        return jnp.where(lane == r, rs, acc)

    outv[...] = lax.fori_loop(0, _ROWS_PER_WORKER, do_row, zer_f)
    pltpu.sync_copy(outv, out_hbm.at[w])


def _sc_select(lflat):
    mesh = plsc.VectorSubcoreMesh(core_axis_name="c", subcore_axis_name="s")
    return pl.kernel(
        _sc_select_body,
        out_type=jax.ShapeDtypeStruct((32, 16), jnp.float32),
        mesh=mesh,
        scratch_types=[
            pltpu.VMEM((_CHUNK,), jnp.float32),
            pltpu.VMEM((_CHUNK,), jnp.float32),
            pltpu.VMEM(((_NBINS + 1) * 16,), jnp.int32),
            pltpu.VMEM(((_NBINS + 1) * 16,), jnp.float32),
            pltpu.VMEM((16,), jnp.float32),
            pltpu.SemaphoreType.DMA,
            pltpu.SemaphoreType.DMA,
        ],
        compiler_params=pltpu.CompilerParams(needs_layout_passes=False),
        interpret=_INTERPRET,
    )(lflat)


def kernel(y_pred, y_true):
    y_pred = y_pred.astype(jnp.float32)
    half = y_pred.shape[0] // 2
    c = y_pred.shape[1]
    rb = _ROWS_PER_BLOCK
    nb = half // rb
    x = y_pred[:half]
    z = y_pred[half:]
    t = jnp.broadcast_to(
        y_true[half:].astype(jnp.int32)[:, None], (half, 128))

    soft = jnp.float32(0.0)
    hard = jnp.float32(0.0)
    hr = half // 2
    nbs = hr // rb
    for s in range(2):
        losses = pl.pallas_call(
            _loss_body,
            grid=(nbs,),
            in_specs=[
                pl.BlockSpec((rb, c), lambda i: (i, 0)),
                pl.BlockSpec((rb, c), lambda i: (i, 0)),
                pl.BlockSpec((8, 128), lambda i: (i, 0)),
            ],
            out_specs=pl.BlockSpec((2 * rb, c), lambda i: (i, 0)),
            out_shape=jax.ShapeDtypeStruct((2 * hr, c), jnp.float32),
            interpret=bool(_INTERPRET),
        )(x[s * hr:(s + 1) * hr], z[s * hr:(s + 1) * hr],
          t[s * hr:(s + 1) * hr])
        sums = _sc_select(losses.reshape(-1))
        per_row = sums[:, :_ROWS_PER_WORKER].reshape(nbs, 2, rb)
        soft = soft + jnp.sum(per_row[:, 0, :])
        hard = hard + jnp.sum(per_row[:, 1, :])
    denom = float(half * _K)
    return soft / denom + _HARD_WEIGHT * (hard / denom)


# speedup vs baseline: 1.1712x; 1.1712x over previous
import functools
import math

import jax
import jax.numpy as jnp
from jax import lax
from jax.experimental import pallas as pl
from jax.experimental.pallas import tpu as pltpu
from jax.experimental.pallas import tpu_sc as plsc

_EPS = 1e-6
_LOG_EPS = math.log(_EPS)
_LOG_1MEPS = math.log1p(-_EPS)
_K = 1000
_HARD_WEIGHT = 0.01
_INTERPRET = False

_ROWS_PER_BLOCK = 8
_N = 100000
_CHUNK = 10000
_NCHUNK = _N // _CHUNK
_NBINS = 1024
_ROWS_PER_WORKER = 4

_C0 = _EPS * _LOG_EPS + (1.0 - _EPS) * _LOG_1MEPS


def _loss_body(x_ref, z_ref, t_ref, out_ref):
    x = x_ref[...]
    z = z_ref[...]
    t = t_ref[...][:, :1]
    ux = jnp.exp(-jnp.abs(x))
    sp_x = jnp.maximum(x, 0.0) + jnp.log1p(ux)
    uz = jnp.exp(-jnp.abs(z))
    sp_z = jnp.maximum(z, 0.0) + jnp.log1p(uz)
    vz = 1.0 / (1.0 + uz)
    s_z = jnp.where(z >= 0.0, vz, uz * vz)
    ls = sp_x - sp_z + s_z * (z - x)
    col = lax.broadcasted_iota(jnp.int32, x.shape, 1)
    lh = (_C0 + sp_x - _EPS * x) - jnp.where(
        col == t, (1.0 - 2.0 * _EPS) * x, 0.0)
    out_ref[...] = jnp.concatenate(
        [jnp.abs(jnp.maximum(ls, 0.0)), jnp.abs(jnp.maximum(lh, 0.0))],
        axis=0)


def _sc_select_body(l_hbm, out_hbm, buf0, buf1, cnt_h, sum_h, outv,
                    sem0, sem1):
    w = lax.axis_index("s") * 2 + lax.axis_index("c")
    lane = lax.iota(jnp.int32, 16)
    ones_i = jnp.ones((16,), jnp.int32)
    zer_i = jnp.zeros((16,), jnp.int32)
    zer_f = jnp.zeros((16,), jnp.float32)

    def hist_pass(base, b_hi):
        def clr(i):
            cnt_h[pl.ds(i * 16, 16)] = zer_i
            if b_hi is not None:
                sum_h[pl.ds(i * 16, 16)] = zer_f

        plsc.parallel_loop(0, _NBINS + 1, unroll=8)(clr)

        bufs = (buf0, buf1)
        sems = (sem0, sem1)
        handles = [None] * _NCHUNK
        handles[0] = pltpu.async_copy(
            l_hbm.at[pl.ds(base, _CHUNK)], bufs[0], sems[0])
        for c in range(_NCHUNK):
            if c + 1 < _NCHUNK:
                handles[c + 1] = pltpu.async_copy(
                    l_hbm.at[pl.ds(base + (c + 1) * _CHUNK, _CHUNK)],
                    bufs[(c + 1) % 2], sems[(c + 1) % 2])
            handles[c].wait()
            cur = bufs[c % 2]

            def scat(j):
                v = cur[pl.ds(j * 16, 16)]
                bits = lax.bitcast_convert_type(v, jnp.int32)
                if b_hi is None:
                    bn = lax.shift_right_logical(bits, 21) & 0x3FF
                    plsc.addupdate_scatter(cnt_h, [bn * 16 + lane], ones_i)
                else:
                    b1v = lax.shift_right_logical(bits, 21) & 0x3FF
                    m_eq = b1v == b_hi
                    m_hi = b1v > b_hi
                    bn = lax.shift_right_logical(bits, 11) & 0x3FF
                    bx = jnp.where(m_hi, jnp.int32(_NBINS), bn)
                    idx = bx * 16 + lane
                    plsc.addupdate_scatter(cnt_h, [idx], ones_i, mask=m_eq)
                    plsc.addupdate_scatter(sum_h, [idx], v,
                                           mask=m_eq | m_hi)

            plsc.parallel_loop(0, _CHUNK // 16, unroll=8)(scat)

    def group_tot(g):
        def acc(i, carry):
            cc, ss = carry
            off = (g * 16 + i) * 16
            return (cc + cnt_h[pl.ds(off, 16)], ss + sum_h[pl.ds(off, 16)])

        cc, ss = lax.fori_loop(0, 16, acc, (zer_i, zer_f))
        return jnp.sum(cc), jnp.sum(ss)

    def bin_tot(b):
        off = b * 16
        return jnp.sum(cnt_h[pl.ds(off, 16)]), jnp.sum(sum_h[pl.ds(off, 16)])

    def find_bin(k_rem):
        def gcond(st):
            g, ca, sa, cg, sg = st
            return (g > 0) & (ca + cg < k_rem)

        def gbody(st):
            g, ca, sa, cg, sg = st
            cn, sn = group_tot(g - 1)
            return (g - 1, ca + cg, sa + sg, cn, sn)

        cg0, sg0 = group_tot(63)
        g, ca, sa, cg, sg = lax.while_loop(
            gcond, gbody, (jnp.int32(63), jnp.int32(0), jnp.float32(0.0),
                           cg0, sg0))

        def bcond(st):
            b, ca, sa, cb, sb = st
            return (b > 0) & (ca + cb < k_rem)

        def bbody(st):
            b, ca, sa, cb, sb = st
            cn, sn = bin_tot(g * 16 + b - 1)
            return (b - 1, ca + cb, sa + sb, cn, sn)

        cb0, sb0 = bin_tot(g * 16 + 15)
        b, ca, sa, cb, sb = lax.while_loop(
            bcond, bbody, (jnp.int32(15), ca, sa, cb0, sb0))
        return g * 16 + b, ca, sa

    def do_row(r, acc):
        base = (w * _ROWS_PER_WORKER + r) * _N
        hist_pass(base, None)
        b1, c1, _ = find_bin(jnp.int32(_K))
        k2 = jnp.int32(_K) - c1
        hist_pass(base, b1)
        b2, c2, s2 = find_bin(k2)
        s_hi = jnp.sum(sum_h[pl.ds(_NBINS * 16, 16)])
        t_bits = (b1 << 21) | (b2 << 11) | (1 << 10)
        t_hat = jnp.max(lax.bitcast_convert_type(
            jnp.full((16,), t_bits, jnp.int32), jnp.float32))
        rs = s_hi + s2 + (k2 - c2).astype(jnp.float32) * t_hat
        return jnp.where(lane == r, rs, acc)

    outv[...] = lax.fori_loop(0, _ROWS_PER_WORKER, do_row, zer_f)
    pltpu.sync_copy(outv, out_hbm.at[w])


def _sc_select(lflat):
    mesh = plsc.VectorSubcoreMesh(core_axis_name="c", subcore_axis_name="s")
    return pl.kernel(
        _sc_select_body,
        out_type=jax.ShapeDtypeStruct((32, 16), jnp.float32),
        mesh=mesh,
        scratch_types=[
            pltpu.VMEM((_CHUNK,), jnp.float32),
            pltpu.VMEM((_CHUNK,), jnp.float32),
            pltpu.VMEM(((_NBINS + 1) * 16,), jnp.int32),
            pltpu.VMEM(((_NBINS + 1) * 16,), jnp.float32),
            pltpu.VMEM((16,), jnp.float32),
            pltpu.SemaphoreType.DMA,
            pltpu.SemaphoreType.DMA,
        ],
        compiler_params=pltpu.CompilerParams(needs_layout_passes=False),
        interpret=_INTERPRET,
    )(lflat)


def kernel(y_pred, y_true):
    y_pred = y_pred.astype(jnp.float32)
    half = y_pred.shape[0] // 2
    c = y_pred.shape[1]
    rb = _ROWS_PER_BLOCK
    nb = half // rb
    x = y_pred[:half]
    z = y_pred[half:]
    t = jnp.broadcast_to(
        y_true[half:].astype(jnp.int32)[:, None], (half, 128))

    losses = pl.pallas_call(
        _loss_body,
        grid=(nb,),
        in_specs=[
            pl.BlockSpec((rb, c), lambda i: (i, 0)),
            pl.BlockSpec((rb, c), lambda i: (i, 0)),
            pl.BlockSpec((8, 128), lambda i: (i, 0)),
        ],
        out_specs=pl.BlockSpec((2 * rb, c), lambda i: (i, 0)),
        out_shape=jax.ShapeDtypeStruct((2 * half, c), jnp.float32),
        interpret=bool(_INTERPRET),
    )(x, z, t)

    sums = _sc_select(losses.reshape(-1))
    per_row = sums[:, :_ROWS_PER_WORKER].reshape(nb, 2, rb)
    denom = float(half * _K)
    soft = jnp.sum(per_row[:, 0, :]) / denom
    hard = jnp.sum(per_row[:, 1, :]) / denom
    return soft + _HARD_WEIGHT * hard
